# baseline (speedup 1.0000x reference)
import functools

import jax
import jax.numpy as jnp
from jax import lax
from jax.experimental import pallas as pl
from jax.experimental.pallas import tpu as pltpu
from jax.experimental.pallas import tpu_sc as plsc

_B = 4096 * 200
_DC = 128
_DS = 32
_CHUNK = 64
_NW = 32
_PER_W = _B // _NW
_NCH = _PER_W // _CHUNK
_NB = 3
_GA = 1
_WD = 2


def kernel(x, x_role, content_table, struct_table):
    x_flat = x.reshape(-1).astype(jnp.int32)
    role_flat = x_role.reshape(-1).astype(jnp.int32)

    mesh = plsc.VectorSubcoreMesh(core_axis_name="c", subcore_axis_name="s")

    @functools.partial(
        pl.kernel,
        mesh=mesh,
        out_type=[
            jax.ShapeDtypeStruct((_B, _DC + _DS), jnp.float32),
            jax.ShapeDtypeStruct((_B,), jnp.int32),
        ],
        scratch_types=[
            pltpu.VMEM((_PER_W,), jnp.int32),
            pltpu.VMEM((_PER_W,), jnp.int32),
            pltpu.VMEM((4 * _DS,), jnp.float32),
            pltpu.VMEM_SHARED((16, _NB, _CHUNK, _DC + _DS), jnp.float32),
            [pltpu.VMEM((_CHUNK, _DC), jnp.float32) for _ in range(_NB)],
            [pltpu.VMEM((_CHUNK, _DS), jnp.float32) for _ in range(_NB)],
            [pltpu.VMEM((_CHUNK,), jnp.int32) for _ in range(_NB)],
            pltpu.SemaphoreType.DMA,
            [pltpu.SemaphoreType.DMA for _ in range(_NB)],
            [pltpu.SemaphoreType.DMA for _ in range(_NB)],
            [pltpu.SemaphoreType.DMA for _ in range(_NB)],
        ],
        compiler_params=pltpu.CompilerParams(use_tc_tiling_on_sc=False,
                                             needs_layout_passes=False),
    )
    def run(x_hbm, role_hbm, ct_hbm, st_hbm, out_hbm, mask_hbm,
            idx_all, role_all, stv, asm_s, content_b, struct_b, mask_b,
            sem_idx, gat_s, wr_s, loc_s):
        wid = lax.axis_index("s") * 2 + lax.axis_index("c")
        sid = lax.axis_index("s")
        w_base = wid * _PER_W
        last = _NCH - 1

        cp_i = pltpu.async_copy(x_hbm.at[pl.ds(w_base, _PER_W)], idx_all,
                                sem_idx)
        cp_r = pltpu.async_copy(role_hbm.at[pl.ds(w_base, _PER_W)], role_all,
                                sem_idx)
        cp_s = [pltpu.async_copy(st_hbm.at[r],
                                 stv.at[pl.ds(r * _DS, _DS)], sem_idx)
                for r in range(4)]
        cp_i.wait()
        cp_r.wait()
        for cp in cp_s:
            cp.wait()

        def gather_halves(c, k):
            off = c * _CHUNK
            h = _CHUNK // 2
            return (
                pltpu.make_async_copy(
                    ct_hbm.at[idx_all.at[pl.ds(off, h)]],
                    content_b[k].at[pl.ds(0, h)], gat_s[k]),
                pltpu.make_async_copy(
                    ct_hbm.at[idx_all.at[pl.ds(off + h, h)]],
                    content_b[k].at[pl.ds(h, h)], gat_s[k]),
            )

        def writes(c, k):
            base = w_base + c * _CHUNK
            return (
                pltpu.make_async_copy(
                    asm_s.at[sid, k], out_hbm.at[pl.ds(base, _CHUNK)],
                    wr_s[k]),
                pltpu.make_async_copy(
                    mask_b[k], mask_hbm.at[pl.ds(base, _CHUNK)], wr_s[k]),
            )

        lanes = jax.lax.iota(jnp.int32, 16)
        n_slots = (_NCH + _NB - 1) // _NB * _NB
        n_iter = n_slots // _NB

        for s in range(_GA):
            for d in gather_halves(jnp.int32(s), s % _NB):
                d.start()

        def step(j, carry):
            for k in range(_NB):
                s = j * _NB + k
                c = jnp.minimum(s, last)
                off = c * _CHUNK

                @pl.when(s >= _WD)
                def _():
                    cw = jnp.minimum(s - _WD, last)
                    for d in writes(cw, (k - _WD) % _NB):
                        d.wait()

                @pl.when(s + _GA <= n_slots - 1)
                def _():
                    cg = jnp.minimum(s + _GA, last)
                    for d in gather_halves(cg, (k + _GA) % _NB):
                        d.start()

                def group(g, carry2):
                    v = idx_all[pl.ds(off + g * 16, 16)]
                    mask_b[k][pl.ds(g * 16, 16)] = jnp.where(
                        v != 0, jnp.int32(1), jnp.int32(0))
                    r16 = role_all[pl.ds(off + g * 16, 16)]
                    rbase = r16 * _DS
                    rows = lanes + g * 16
                    for col in range(_DS):
                        j16 = jnp.full((16,), col, jnp.int32)
                        vals = plsc.load_gather(stv, [rbase + col])
                        plsc.store_scatter(struct_b[k], [rows, j16], vals)
                    return carry2

                lax.fori_loop(0, _CHUNK // 16, group, 0)

                for d in gather_halves(c, k):
                    d.wait()
                p1 = pltpu.async_copy(
                    content_b[k], asm_s.at[sid, k, :, pl.ds(0, _DC)],
                    loc_s[k])
                p2 = pltpu.async_copy(
                    struct_b[k], asm_s.at[sid, k, :, pl.ds(_DC, _DS)],
                    loc_s[k])
                p1.wait()
                p2.wait()
                for d in writes(c, k):
                    d.start()
            return carry

        lax.fori_loop(0, n_iter, step, 0)

        for s in range(n_slots - _WD, n_slots):
            c = jnp.minimum(jnp.int32(s), last)
            for d in writes(c, s % _NB):
                d.wait()

    out, mask_i32 = run(x_flat, role_flat, content_table, struct_table)
    ret = out.reshape(x.shape[0], x.shape[1], _DC + _DS)
    mask = mask_i32.reshape(x.shape).astype(bool)
    return (ret, mask)


# speedup vs baseline: 1.0753x; 1.0753x over previous
import functools

import jax
import jax.numpy as jnp
from jax import lax
from jax.experimental import pallas as pl
from jax.experimental.pallas import tpu as pltpu
from jax.experimental.pallas import tpu_sc as plsc

_B = 4096 * 200
_DC = 128
_DS = 32
_CHUNK = 128
_NW = 32
_PER_W = _B // _NW
_NCH = _PER_W // _CHUNK
_NB = 3
_GA = 1
_WD = 2


def kernel(x, x_role, content_table, struct_table):
    x_flat = x.reshape(-1).astype(jnp.int32)
    role_flat = x_role.reshape(-1).astype(jnp.int32)

    mesh = plsc.VectorSubcoreMesh(core_axis_name="c", subcore_axis_name="s")

    @functools.partial(
        pl.kernel,
        mesh=mesh,
        out_type=[
            jax.ShapeDtypeStruct((_B, _DC + _DS), jnp.float32),
            jax.ShapeDtypeStruct((_B,), jnp.int32),
        ],
        scratch_types=[
            pltpu.VMEM((_PER_W,), jnp.int32),
            pltpu.VMEM((_PER_W,), jnp.int32),
            pltpu.VMEM((4 * _DS,), jnp.float32),
            [pltpu.VMEM((_CHUNK, _DC), jnp.float32) for _ in range(_NB)],
            [pltpu.VMEM((_CHUNK, _DS), jnp.float32) for _ in range(_NB)],
            [pltpu.VMEM((_CHUNK,), jnp.int32) for _ in range(_NB)],
            pltpu.SemaphoreType.DMA,
            [pltpu.SemaphoreType.DMA for _ in range(_NB)],
            [pltpu.SemaphoreType.DMA for _ in range(_NB)],
        ],
        compiler_params=pltpu.CompilerParams(use_tc_tiling_on_sc=False,
                                             needs_layout_passes=False),
    )
    def run(x_hbm, role_hbm, ct_hbm, st_hbm, out_hbm, mask_hbm,
            idx_all, role_all, stv, content_b, struct_b, mask_b,
            sem_idx, gat_s, wr_s):
        wid = lax.axis_index("s") * 2 + lax.axis_index("c")
        w_base = wid * _PER_W
        last = _NCH - 1

        cp_i = pltpu.async_copy(x_hbm.at[pl.ds(w_base, _PER_W)], idx_all,
                                sem_idx)
        cp_r = pltpu.async_copy(role_hbm.at[pl.ds(w_base, _PER_W)], role_all,
                                sem_idx)
        cp_s = [pltpu.async_copy(st_hbm.at[r],
                                 stv.at[pl.ds(r * _DS, _DS)], sem_idx)
                for r in range(4)]
        cp_i.wait()
        cp_r.wait()
        for cp in cp_s:
            cp.wait()

        def gather(c, k):
            off = c * _CHUNK
            return pltpu.make_async_copy(
                ct_hbm.at[idx_all.at[pl.ds(off, _CHUNK)]],
                content_b[k], gat_s[k])

        def writes(c, k):
            base = w_base + c * _CHUNK
            return (
                pltpu.make_async_copy(
                    content_b[k],
                    out_hbm.at[pl.ds(base, _CHUNK), pl.ds(0, _DC)], wr_s[k]),
                pltpu.make_async_copy(
                    struct_b[k],
                    out_hbm.at[pl.ds(base, _CHUNK), pl.ds(_DC, _DS)], wr_s[k]),
                pltpu.make_async_copy(
                    mask_b[k], mask_hbm.at[pl.ds(base, _CHUNK)], wr_s[k]),
            )

        lanes = jax.lax.iota(jnp.int32, 16)
        n_slots = (_NCH + _NB - 1) // _NB * _NB
        n_iter = n_slots // _NB

        for s in range(_GA):
            gather(jnp.int32(s), s % _NB).start()

        def step(j, carry):
            for k in range(_NB):
                s = j * _NB + k
                c = jnp.minimum(s, last)
                off = c * _CHUNK

                @pl.when(s >= _WD)
                def _():
                    cw = jnp.minimum(s - _WD, last)
                    for d in writes(cw, (k - _WD) % _NB):
                        d.wait()

                @pl.when(s + _GA <= n_slots - 1)
                def _():
                    cg = jnp.minimum(s + _GA, last)
                    gather(cg, (k + _GA) % _NB).start()

                def group(g, carry2):
                    v = idx_all[pl.ds(off + g * 16, 16)]
                    mask_b[k][pl.ds(g * 16, 16)] = jnp.where(
                        v != 0, jnp.int32(1), jnp.int32(0))
                    r16 = role_all[pl.ds(off + g * 16, 16)]
                    rbase = r16 * _DS
                    rows = lanes + g * 16
                    for col in range(_DS):
                        j16 = jnp.full((16,), col, jnp.int32)
                        vals = plsc.load_gather(stv, [rbase + col])
                        plsc.store_scatter(struct_b[k], [rows, j16], vals)
                    return carry2

                lax.fori_loop(0, _CHUNK // 16, group, 0)

                gather(c, k).wait()
                for d in writes(c, k):
                    d.start()
            return carry

        lax.fori_loop(0, n_iter, step, 0)

        for s in range(n_slots - _WD, n_slots):
            c = jnp.minimum(jnp.int32(s), last)
            for d in writes(c, s % _NB):
                d.wait()

    out, mask_i32 = run(x_flat, role_flat, content_table, struct_table)
    ret = out.reshape(x.shape[0], x.shape[1], _DC + _DS)
    mask = mask_i32.reshape(x.shape).astype(bool)
    return (ret, mask)
